# baseline (speedup 1.0000x reference)
import functools

import jax
import jax.numpy as jnp
from jax import lax
from jax.experimental import pallas as pl
from jax.experimental.pallas import tpu as pltpu
from jax.experimental.pallas import tpu_sc as plsc

_D = 32
_PACK = 4
_CH = 128
_NB = 5
_L = 16


@functools.cache
def _make_lookup(B):
    info = plsc.get_sparse_core_info()
    nc = info.num_cores
    nw = nc * info.num_subcores
    b_per_w = B // nw
    n_dma = b_per_w // _CH
    n_body = n_dma // _NB
    assert n_body * _NB == n_dma
    ch128 = _CH // _PACK
    mesh = plsc.VectorSubcoreMesh(core_axis_name="c", subcore_axis_name="s")

    @functools.partial(
        pl.kernel,
        mesh=mesh,
        compiler_params=pltpu.CompilerParams(
            use_tc_tiling_on_sc=True, needs_layout_passes=False
        ),
        out_type=jax.ShapeDtypeStruct((B // _PACK, _PACK * _D), jnp.float32),
        scratch_types=[
            pltpu.VMEM((n_dma, _CH), jnp.int32),
            pltpu.VMEM((n_dma, _CH), jnp.int32),
            pltpu.VMEM((_NB, _CH, _PACK * _D), jnp.float32),
            pltpu.VMEM((_NB, _CH // _PACK, _PACK * _D), jnp.float32),
            pltpu.SemaphoreType.DMA((_NB,)),
            pltpu.SemaphoreType.DMA((_NB,)),
        ],
    )
    def k(t128, idx_hbm, out_hbm, ex_v, hi_v, g_v, s_v, gsem, ssem):
        wid = lax.axis_index("s") * nc + lax.axis_index("c")
        pltpu.sync_copy(idx_hbm.at[wid], ex_v)

        lane = lax.iota(jnp.int32, _L)

        def prep(j, carry):
            def prep16(kk, carry2):
                v = ex_v[j, pl.ds(kk * _L, _L)]
                hi_v[j, pl.ds(kk * _L, _L)] = lax.shift_right_logical(v, 2)
                ex_v[j, pl.ds(kk * _L, _L)] = _D * (v & 3)
                return carry2

            return lax.fori_loop(0, _CH // _L, prep16, carry)

        lax.fori_loop(0, n_dma, prep, 0)

        base128 = pl.multiple_of(wid * (b_per_w // _PACK), ch128)

        def fire(c, slot):
            return pltpu.async_copy(
                t128.at[hi_v.at[c]], g_v.at[slot], gsem.at[slot]
            )

        def process(c, slot, first, fire_next):
            pltpu.make_async_copy(
                t128.at[hi_v.at[c]], g_v.at[slot], gsem.at[slot]
            ).wait()
            if not first:
                pltpu.make_async_copy(
                    s_v.at[slot],
                    out_hbm.at[pl.ds(base128, ch128)],
                    ssem.at[slot],
                ).wait()
            g2 = g_v.at[slot]
            s2 = s_v.at[slot]

            def extract16(kk, carry2):
                rv = _L * kk + lane
                prow = lax.shift_right_logical(rv, 2)
                pcol = _D * (rv & 3)
                fv = ex_v[c, pl.ds(kk * _L, _L)]
                for col in range(_D):
                    vals = plsc.load_gather(g2, [rv, fv + col])
                    plsc.store_scatter(s2, [prow, pcol + col], vals)
                return carry2

            lax.fori_loop(0, _CH // _L, extract16, 0)
            pltpu.async_copy(
                s_v.at[slot],
                out_hbm.at[pl.ds(base128 + c * ch128, ch128)],
                ssem.at[slot],
            )
            if fire_next:
                fire(c + _NB, slot)

        for i in range(_NB):
            fire(i, i)
        for i in range(_NB):
            process(i, i, first=True, fire_next=True)

        def body(t, carry):
            for i in range(_NB):
                process(t * _NB + i, i, first=False, fire_next=True)
            return carry

        lax.fori_loop(1, n_body - 1, body, 0)

        for i in range(_NB):
            process((n_body - 1) * _NB + i, i, first=False, fire_next=False)
        for i in range(_NB):
            pltpu.make_async_copy(
                s_v.at[i],
                out_hbm.at[pl.ds(base128, ch128)],
                ssem.at[i],
            ).wait()

    return k


def kernel(input, table):
    B = input.size
    info = plsc.get_sparse_core_info()
    nw = info.num_cores * info.num_subcores
    t128 = table.reshape(table.shape[0] // _PACK, _PACK * _D)
    idx3d = input.reshape(nw, B // (nw * _CH), _CH)
    out = _make_lookup(B)(t128, idx3d)
    return out.reshape(input.shape + (_D,))


# speedup vs baseline: 1.3541x; 1.3541x over previous
import functools

import jax
import jax.numpy as jnp
from jax import lax
from jax.experimental import pallas as pl
from jax.experimental.pallas import tpu as pltpu
from jax.experimental.pallas import tpu_sc as plsc

_D = 32
_NB = 8


@functools.cache
def _make_lookup(BATCH, SEQ):
    info = plsc.get_sparse_core_info()
    nc = info.num_cores
    nw = nc * info.num_subcores
    rows_w = BATCH // nw
    n_body = rows_w // _NB
    assert n_body * _NB == rows_w and n_body >= 2
    mesh = plsc.VectorSubcoreMesh(core_axis_name="c", subcore_axis_name="s")

    @functools.partial(
        pl.kernel,
        mesh=mesh,
        compiler_params=pltpu.CompilerParams(use_tc_tiling_on_sc=False),
        out_type=jax.ShapeDtypeStruct((BATCH, SEQ, _D), jnp.float32),
        scratch_types=[
            pltpu.VMEM((rows_w, SEQ), jnp.int32),
            pltpu.VMEM((_NB, SEQ, _D), jnp.float32),
            pltpu.SemaphoreType.DMA((_NB,)),
            pltpu.SemaphoreType.DMA((_NB,)),
        ],
    )
    def k(table_hbm, idx_hbm, out_hbm, idx_v, rows_v, gsem, ssem):
        wid = lax.axis_index("s") * nc + lax.axis_index("c")
        base = pl.multiple_of(wid * rows_w, _NB)
        pltpu.sync_copy(idx_hbm.at[pl.ds(base, rows_w)], idx_v)

        def fire(c, slot):
            return pltpu.async_copy(
                table_hbm.at[idx_v.at[c]], rows_v.at[slot], gsem.at[slot]
            )

        def process(c, slot, first, fire_next):
            pltpu.make_async_copy(
                table_hbm.at[idx_v.at[c]], rows_v.at[slot], gsem.at[slot]
            ).wait()
            if not first:
                pltpu.make_async_copy(
                    rows_v.at[slot], out_hbm.at[base], ssem.at[slot]
                ).wait()
            pltpu.async_copy(
                rows_v.at[slot], out_hbm.at[base + c], ssem.at[slot]
            )
            if fire_next:
                fire(c + _NB, slot)

        for i in range(_NB):
            fire(i, i)
        for i in range(_NB):
            process(i, i, first=True, fire_next=True)

        def body(t, carry):
            for i in range(_NB):
                process(t * _NB + i, i, first=False, fire_next=True)
            return carry

        lax.fori_loop(1, n_body - 1, body, 0)

        for i in range(_NB):
            process((n_body - 1) * _NB + i, i, first=False, fire_next=False)
        for i in range(_NB):
            pltpu.make_async_copy(
                rows_v.at[i], out_hbm.at[base], ssem.at[i]
            ).wait()

    return k


def kernel(input, table):
    return _make_lookup(*input.shape)(table, input)
